# Initial kernel scaffold; baseline (speedup 1.0000x reference)
import jax
import jax.numpy as jnp
from jax.experimental import pallas as pl


def kernel(X, y):
    raise NotImplementedError("write your pallas kernel here")


# speedup vs baseline: 32.1735x; 32.1735x over previous
import jax
import jax.numpy as jnp
from jax.experimental import pallas as pl
from jax.experimental.pallas import tpu as pltpu

_N = 4096
_D = 512
_K = 3
_BLK = 256
_NBLK = _N // _BLK
_BIG = 1e30


def _digamma(x):
    acc = jnp.zeros_like(x)
    for _ in range(6):
        acc = acc - 1.0 / x
        x = x + 1.0
    inv = 1.0 / x
    inv2 = inv * inv
    series = (jnp.log(x) - 0.5 * inv
              - inv2 * ((1.0 / 12.0) - inv2 * ((1.0 / 120.0) - inv2 * (1.0 / 252.0))))
    return series + acc


def _mi_block_kernel(xb_ref, xf_ref, ycol_ref, yrow_ref, out_ref):
    xb = xb_ref[...]
    xf = xf_ref[...]

    dot = jax.lax.dot_general(xb, xf, (((1,), (1,)), ((), ())),
                              preferred_element_type=jnp.float32)
    sqb = jnp.sum(xb * xb, axis=1, keepdims=True)
    ones_row = jnp.ones((1, _D), jnp.float32)
    sqf = jax.lax.dot_general(ones_row, xf * xf, (((1,), (1,)), ((), ())),
                              preferred_element_type=jnp.float32)
    d2 = jnp.maximum(sqb + sqf - 2.0 * dot, 0.0)

    same = ycol_ref[...] == yrow_ref[...]
    class_count = jnp.sum(jnp.where(same, 1.0, 0.0), axis=1, keepdims=True)
    w = jnp.where(same, d2, _BIG)

    cur = jnp.full((_BLK, 1), -1.0, jnp.float32)
    cnt = jnp.zeros((_BLK, 1), jnp.float32)
    anchor = jnp.full((_BLK, 1), _BIG, jnp.float32)
    for _ in range(_K + 1):
        nxt = jnp.min(jnp.where(w > cur, w, _BIG), axis=1, keepdims=True)
        c = jnp.sum(jnp.where(w == nxt, 1.0, 0.0), axis=1, keepdims=True)
        take = cnt < (_K + 1)
        anchor = jnp.where(take, nxt, anchor)
        cnt = cnt + jnp.where(take, c, 0.0)
        cur = nxt

    m = jnp.sum(jnp.where(d2 <= anchor, 1.0, 0.0), axis=1, keepdims=True) - 1.0
    dig_m = _digamma(m + 1e-7)
    dig_s = _digamma(class_count)

    partial = jnp.sum(jnp.concatenate([dig_m, dig_s], axis=1),
                      axis=0, keepdims=True)
    out_ref[...] = partial.reshape(1, 1, 2)


def kernel(X, y):
    yf = y.astype(jnp.float32)
    ycol = yf.reshape(_N, 1)
    yrow = yf.reshape(1, _N)

    partials = pl.pallas_call(
        _mi_block_kernel,
        grid=(_NBLK,),
        in_specs=[
            pl.BlockSpec((_BLK, _D), lambda i: (i, 0)),
            pl.BlockSpec((_N, _D), lambda i: (0, 0)),
            pl.BlockSpec((_BLK, 1), lambda i: (i, 0)),
            pl.BlockSpec((1, _N), lambda i: (0, 0)),
        ],
        out_specs=pl.BlockSpec((1, 1, 2), lambda i: (i, 0, 0)),
        out_shape=jax.ShapeDtypeStruct((_NBLK, 1, 2), jnp.float32),
        compiler_params=pltpu.CompilerParams(
            dimension_semantics=("parallel",)),
    )(X, X, ycol, yrow)

    sums = jnp.sum(partials, axis=(0, 1))
    avg_m = sums[0] / _N
    avg_nx = sums[1] / _N
    dig_n = jax.scipy.special.digamma(jnp.float32(_N))
    dig_k = jax.scipy.special.digamma(jnp.float32(_K))
    mi = (dig_n - avg_nx + dig_k - avg_m) / jnp.log(jnp.float32(2.0))
    return jax.nn.relu(mi)


# speedup vs baseline: 35.9995x; 1.1189x over previous
import jax
import jax.numpy as jnp
from jax.experimental import pallas as pl
from jax.experimental.pallas import tpu as pltpu

_N = 4096
_D = 512
_NCLS = 10
_K = 3
_BLK = 256
_NBLK = _N // _BLK
_BIG = 1e30


def _digamma(x):
    acc = jnp.zeros_like(x)
    for _ in range(6):
        acc = acc - 1.0 / x
        x = x + 1.0
    inv = 1.0 / x
    inv2 = inv * inv
    series = (jnp.log(x) - 0.5 * inv
              - inv2 * ((1.0 / 12.0) - inv2 * ((1.0 / 120.0) - inv2 * (1.0 / 252.0))))
    return series + acc


def _mi_block_kernel(xb_ref, xf_ref, ycol_ref, yrow_ref, out_ref, sqf_ref):
    step = pl.program_id(0)

    @pl.when(step == 0)
    def _prologue():
        xf = xf_ref[...]
        sqf_ref[...] = jax.lax.dot_general(
            jnp.ones((1, _D), jnp.float32), xf * xf, (((1,), (1,)), ((), ())),
            preferred_element_type=jnp.float32)

    xb = xb_ref[...]
    dotm2 = jax.lax.dot_general(xb * -2.0, xf_ref[...], (((1,), (1,)), ((), ())),
                                preferred_element_type=jnp.float32)
    sqb = jnp.sum(xb * xb, axis=1, keepdims=True)
    d2 = jnp.maximum((sqb + sqf_ref[...]) + dotm2, 0.0)

    yrow = yrow_ref[...]
    same = ycol_ref[...] == yrow
    w = jnp.where(same, d2, _BIG)

    v1 = jnp.min(w, axis=1, keepdims=True)
    cnt = jnp.sum(jnp.where(w == v1, 1.0, 0.0), axis=1, keepdims=True)
    anchor = v1
    cur = v1
    for _ in range(_K):
        nxt = jnp.min(jnp.where(w > cur, w, _BIG), axis=1, keepdims=True)
        c = jnp.sum(jnp.where(w == nxt, 1.0, 0.0), axis=1, keepdims=True)
        take = cnt < (_K + 1)
        anchor = jnp.where(take, nxt, anchor)
        cnt = cnt + jnp.where(take, c, 0.0)
        cur = nxt

    m = jnp.sum(jnp.where(d2 <= anchor, 1.0, 0.0), axis=1, keepdims=True) - 1.0
    dig_m = jnp.sum(_digamma(m + 1e-7))

    cls_sum = jnp.zeros((), jnp.float32)
    for c in range(_NCLS):
        n_c = jnp.sum(jnp.where(yrow == float(c), 1.0, 0.0))
        cls_sum = cls_sum + n_c * _digamma(n_c)
    dig_s = jnp.where(step == 0, cls_sum, 0.0)

    lane = jax.lax.broadcasted_iota(jnp.int32, (1, 1, 2), 2)
    out_ref[...] = jnp.where(lane == 0, dig_m, dig_s)


def kernel(X, y):
    yf = y.astype(jnp.float32)
    ycol = yf.reshape(_N, 1)
    yrow = yf.reshape(1, _N)

    partials = pl.pallas_call(
        _mi_block_kernel,
        grid=(_NBLK,),
        in_specs=[
            pl.BlockSpec((_BLK, _D), lambda i: (i, 0)),
            pl.BlockSpec((_N, _D), lambda i: (0, 0)),
            pl.BlockSpec((_BLK, 1), lambda i: (i, 0)),
            pl.BlockSpec((1, _N), lambda i: (0, 0)),
        ],
        out_specs=pl.BlockSpec((1, 1, 2), lambda i: (i, 0, 0)),
        out_shape=jax.ShapeDtypeStruct((_NBLK, 1, 2), jnp.float32),
        scratch_shapes=[pltpu.VMEM((1, _N), jnp.float32)],
        compiler_params=pltpu.CompilerParams(
            dimension_semantics=("arbitrary",)),
    )(X, X, ycol, yrow)

    sums = jnp.sum(partials, axis=(0, 1))
    avg_m = sums[0] / _N
    avg_nx = sums[1] / _N
    dig_n = jax.scipy.special.digamma(jnp.float32(_N))
    dig_k = jax.scipy.special.digamma(jnp.float32(_K))
    mi = (dig_n - avg_nx + dig_k - avg_m) / jnp.log(jnp.float32(2.0))
    return jax.nn.relu(mi)
